# baseline (speedup 1.0000x reference)
import math

import jax
import jax.numpy as jnp
from jax.experimental import pallas as pl
from jax.experimental.pallas import tpu as pltpu

_LAMDA = 0.5
_ALPHA = 0.1
_BM = 200


def _entry_kernel(x_ref, w0t_ref, b0_ref, o_ref):
    o_ref[...] = jax.nn.relu(
        jnp.dot(x_ref[...], w0t_ref[...], preferred_element_type=jnp.float32)
        + b0_ref[...]
    )


def _layer_kernel(adj_ref, inner_ref, h0_ref, weff_ref, o_ref):
    hi = jnp.dot(adj_ref[...], inner_ref[...],
                 preferred_element_type=jnp.float32)
    support = (1.0 - _ALPHA) * hi + _ALPHA * h0_ref[...]
    o_ref[...] = jax.nn.relu(
        jnp.dot(support, weff_ref[...], preferred_element_type=jnp.float32))


def kernel(x, dia_len, topicLabel, adj, W0, b0, Wc):
    n, nfeat = x.shape
    nhid = W0.shape[0]
    nlayers = Wc.shape[0]
    bm = _BM if n % _BM == 0 else n
    grid = (n // bm,)

    h0 = pl.pallas_call(
        _entry_kernel,
        grid=grid,
        in_specs=[
            pl.BlockSpec((bm, nfeat), lambda i: (i, 0)),
            pl.BlockSpec((nfeat, nhid), lambda i: (0, 0)),
            pl.BlockSpec((1, nhid), lambda i: (0, 0)),
        ],
        out_specs=pl.BlockSpec((bm, nhid), lambda i: (i, 0)),
        out_shape=jax.ShapeDtypeStruct((n, nhid), jnp.float32),
        compiler_params=pltpu.CompilerParams(
            dimension_semantics=("parallel",)),
    )(x, W0.T, b0.reshape(1, nhid))

    eye = jnp.eye(nhid, dtype=jnp.float32)
    layer_inner = h0
    for i in range(nlayers):
        theta = math.log(_LAMDA / (i + 1) + 1.0)
        w_eff = theta * Wc[i] + (1.0 - theta) * eye
        layer_inner = pl.pallas_call(
            _layer_kernel,
            grid=grid,
            in_specs=[
                pl.BlockSpec((bm, n), lambda i: (i, 0)),
                pl.BlockSpec((n, nhid), lambda i: (0, 0)),
                pl.BlockSpec((bm, nhid), lambda i: (i, 0)),
                pl.BlockSpec((nhid, nhid), lambda i: (0, 0)),
            ],
            out_specs=pl.BlockSpec((bm, nhid), lambda i: (i, 0)),
            out_shape=jax.ShapeDtypeStruct((n, nhid), jnp.float32),
            compiler_params=pltpu.CompilerParams(
                dimension_semantics=("arbitrary",)),
        )(adj, layer_inner, h0, w_eff)
    return layer_inner


# speedup vs baseline: 1.0756x; 1.0756x over previous
import math

import jax
import jax.numpy as jnp
from jax.experimental import pallas as pl
from jax.experimental.pallas import tpu as pltpu

_LAMDA = 0.5
_ALPHA = 0.1
_BM = 200


def _entry_kernel(x_ref, w0t_ref, b0_ref, o_ref, obf_ref):
    h = jax.nn.relu(
        jnp.dot(x_ref[...], w0t_ref[...], preferred_element_type=jnp.float32)
        + b0_ref[...]
    )
    o_ref[...] = h
    obf_ref[...] = h.astype(jnp.bfloat16)


def _layer1_kernel(adj_ref, innerbf_ref, h0_ref, weff_ref, o_ref, adjbf_ref):
    adj_bf = adj_ref[...].astype(jnp.bfloat16)
    adjbf_ref[...] = adj_bf
    hi = jnp.dot(adj_bf, innerbf_ref[...],
                 preferred_element_type=jnp.float32)
    support = (1.0 - _ALPHA) * hi + _ALPHA * h0_ref[...]
    o_ref[...] = jax.nn.relu(
        jnp.dot(support, weff_ref[...], preferred_element_type=jnp.float32)
    ).astype(jnp.bfloat16)


def _layer_kernel(adjbf_ref, innerbf_ref, h0_ref, weff_ref, o_ref):
    hi = jnp.dot(adjbf_ref[...], innerbf_ref[...],
                 preferred_element_type=jnp.float32)
    support = (1.0 - _ALPHA) * hi + _ALPHA * h0_ref[...]
    out = jax.nn.relu(
        jnp.dot(support, weff_ref[...], preferred_element_type=jnp.float32))
    o_ref[...] = out.astype(o_ref.dtype)


def kernel(x, dia_len, topicLabel, adj, W0, b0, Wc):
    n, nfeat = x.shape
    nhid = W0.shape[0]
    nlayers = Wc.shape[0]
    bm = _BM if n % _BM == 0 else n
    grid = (n // bm,)

    h0, h0_bf = pl.pallas_call(
        _entry_kernel,
        grid=grid,
        in_specs=[
            pl.BlockSpec((bm, nfeat), lambda i: (i, 0)),
            pl.BlockSpec((nfeat, nhid), lambda i: (0, 0)),
            pl.BlockSpec((1, nhid), lambda i: (0, 0)),
        ],
        out_specs=[
            pl.BlockSpec((bm, nhid), lambda i: (i, 0)),
            pl.BlockSpec((bm, nhid), lambda i: (i, 0)),
        ],
        out_shape=[
            jax.ShapeDtypeStruct((n, nhid), jnp.float32),
            jax.ShapeDtypeStruct((n, nhid), jnp.bfloat16),
        ],
        compiler_params=pltpu.CompilerParams(
            dimension_semantics=("parallel",)),
    )(x, W0.T, b0.reshape(1, nhid))

    eye = jnp.eye(nhid, dtype=jnp.float32)

    def w_eff(i):
        theta = math.log(_LAMDA / (i + 1) + 1.0)
        return theta * Wc[i] + (1.0 - theta) * eye

    inner_bf, adj_bf = pl.pallas_call(
        _layer1_kernel,
        grid=grid,
        in_specs=[
            pl.BlockSpec((bm, n), lambda i: (i, 0)),
            pl.BlockSpec((n, nhid), lambda i: (0, 0)),
            pl.BlockSpec((bm, nhid), lambda i: (i, 0)),
            pl.BlockSpec((nhid, nhid), lambda i: (0, 0)),
        ],
        out_specs=[
            pl.BlockSpec((bm, nhid), lambda i: (i, 0)),
            pl.BlockSpec((bm, n), lambda i: (i, 0)),
        ],
        out_shape=[
            jax.ShapeDtypeStruct((n, nhid), jnp.bfloat16),
            jax.ShapeDtypeStruct((n, n), jnp.bfloat16),
        ],
        compiler_params=pltpu.CompilerParams(
            dimension_semantics=("arbitrary",)),
    )(adj, h0_bf, h0, w_eff(0))

    for i in range(1, nlayers):
        out_dtype = jnp.float32 if i == nlayers - 1 else jnp.bfloat16
        inner_bf = pl.pallas_call(
            _layer_kernel,
            grid=grid,
            in_specs=[
                pl.BlockSpec((bm, n), lambda i: (i, 0)),
                pl.BlockSpec((n, nhid), lambda i: (0, 0)),
                pl.BlockSpec((bm, nhid), lambda i: (i, 0)),
                pl.BlockSpec((nhid, nhid), lambda i: (0, 0)),
            ],
            out_specs=pl.BlockSpec((bm, nhid), lambda i: (i, 0)),
            out_shape=jax.ShapeDtypeStruct((n, nhid), out_dtype),
            compiler_params=pltpu.CompilerParams(
                dimension_semantics=("arbitrary",)),
        )(adj_bf, inner_bf, h0, w_eff(i))
    return inner_bf


# speedup vs baseline: 1.2172x; 1.1317x over previous
import math

import jax
import jax.numpy as jnp
from jax.experimental import pallas as pl
from jax.experimental.pallas import tpu as pltpu

_LAMDA = 0.5
_ALPHA = 0.1
_BM = 200
_BM2 = 1000


def _entry_kernel(x_ref, w0t_ref, b0_ref, o_ref, obf_ref):
    h = jax.nn.relu(
        jnp.dot(x_ref[...], w0t_ref[...], preferred_element_type=jnp.float32)
        + b0_ref[...]
    )
    o_ref[...] = h
    obf_ref[...] = h.astype(jnp.bfloat16)


def _layer1_kernel(adj_ref, innerbf_ref, h0_ref, weff_ref, o_ref, adjbf_ref):
    adj_bf = adj_ref[...].astype(jnp.bfloat16)
    adjbf_ref[...] = adj_bf
    hi = jnp.dot(adj_bf, innerbf_ref[...],
                 preferred_element_type=jnp.float32)
    support = (1.0 - _ALPHA) * hi + _ALPHA * h0_ref[...]
    o_ref[...] = jax.nn.relu(
        jnp.dot(support, weff_ref[...], preferred_element_type=jnp.float32)
    ).astype(jnp.bfloat16)


def _layer_kernel(adjbf_ref, innerbf_ref, h0_ref, weff_ref, o_ref):
    hi = jnp.dot(adjbf_ref[...], innerbf_ref[...],
                 preferred_element_type=jnp.float32)
    support = (1.0 - _ALPHA) * hi + _ALPHA * h0_ref[...]
    out = jax.nn.relu(
        jnp.dot(support, weff_ref[...], preferred_element_type=jnp.float32))
    o_ref[...] = out.astype(o_ref.dtype)


def kernel(x, dia_len, topicLabel, adj, W0, b0, Wc):
    n, nfeat = x.shape
    nhid = W0.shape[0]
    nlayers = Wc.shape[0]
    bm = _BM if n % _BM == 0 else n
    grid = (n // bm,)

    h0, h0_bf = pl.pallas_call(
        _entry_kernel,
        grid=grid,
        in_specs=[
            pl.BlockSpec((bm, nfeat), lambda i: (i, 0)),
            pl.BlockSpec((nfeat, nhid), lambda i: (0, 0)),
            pl.BlockSpec((1, nhid), lambda i: (0, 0)),
        ],
        out_specs=[
            pl.BlockSpec((bm, nhid), lambda i: (i, 0)),
            pl.BlockSpec((bm, nhid), lambda i: (i, 0)),
        ],
        out_shape=[
            jax.ShapeDtypeStruct((n, nhid), jnp.float32),
            jax.ShapeDtypeStruct((n, nhid), jnp.bfloat16),
        ],
        compiler_params=pltpu.CompilerParams(
            dimension_semantics=("parallel",)),
    )(x, W0.T, b0.reshape(1, nhid))

    eye = jnp.eye(nhid, dtype=jnp.float32)

    def w_eff(i):
        theta = math.log(_LAMDA / (i + 1) + 1.0)
        return theta * Wc[i] + (1.0 - theta) * eye

    inner_bf, adj_bf = pl.pallas_call(
        _layer1_kernel,
        grid=grid,
        in_specs=[
            pl.BlockSpec((bm, n), lambda i: (i, 0)),
            pl.BlockSpec((n, nhid), lambda i: (0, 0)),
            pl.BlockSpec((bm, nhid), lambda i: (i, 0)),
            pl.BlockSpec((nhid, nhid), lambda i: (0, 0)),
        ],
        out_specs=[
            pl.BlockSpec((bm, nhid), lambda i: (i, 0)),
            pl.BlockSpec((bm, n), lambda i: (i, 0)),
        ],
        out_shape=[
            jax.ShapeDtypeStruct((n, nhid), jnp.bfloat16),
            jax.ShapeDtypeStruct((n, n), jnp.bfloat16),
        ],
        compiler_params=pltpu.CompilerParams(
            dimension_semantics=("parallel",)),
    )(adj, h0_bf, h0, w_eff(0))

    bm2 = _BM2 if n % _BM2 == 0 else bm
    grid2 = (n // bm2,)
    for i in range(1, nlayers):
        out_dtype = jnp.float32 if i == nlayers - 1 else jnp.bfloat16
        inner_bf = pl.pallas_call(
            _layer_kernel,
            grid=grid2,
            in_specs=[
                pl.BlockSpec((bm2, n), lambda i: (i, 0)),
                pl.BlockSpec((n, nhid), lambda i: (0, 0)),
                pl.BlockSpec((bm2, nhid), lambda i: (i, 0)),
                pl.BlockSpec((nhid, nhid), lambda i: (0, 0)),
            ],
            out_specs=pl.BlockSpec((bm2, nhid), lambda i: (i, 0)),
            out_shape=jax.ShapeDtypeStruct((n, nhid), out_dtype),
            compiler_params=pltpu.CompilerParams(
                dimension_semantics=("parallel",)),
        )(adj_bf, inner_bf, h0, w_eff(i))
    return inner_bf


# speedup vs baseline: 1.2388x; 1.0177x over previous
import math

import jax
import jax.numpy as jnp
from jax.experimental import pallas as pl
from jax.experimental.pallas import tpu as pltpu

_LAMDA = 0.5
_ALPHA = 0.1
_BM = 400
_BM2 = 400
_BME = 2000


def _entry_kernel(x_ref, w0t_ref, b0_ref, o_ref, obf_ref):
    h = jax.nn.relu(
        jnp.dot(x_ref[...], w0t_ref[...], preferred_element_type=jnp.float32)
        + b0_ref[...]
    )
    o_ref[...] = h
    obf_ref[...] = h.astype(jnp.bfloat16)


def _layer1_kernel(adj_ref, innerbf_ref, h0_ref, weff_ref, o_ref, adjbf_ref):
    adj_bf = adj_ref[...].astype(jnp.bfloat16)
    adjbf_ref[...] = adj_bf
    hi = jnp.dot(adj_bf, innerbf_ref[...],
                 preferred_element_type=jnp.float32)
    support = (1.0 - _ALPHA) * hi + _ALPHA * h0_ref[...]
    o_ref[...] = jax.nn.relu(
        jnp.dot(support, weff_ref[...], preferred_element_type=jnp.float32)
    ).astype(jnp.bfloat16)


def _layer_kernel(adjbf_ref, innerbf_ref, h0_ref, weff_ref, o_ref):
    hi = jnp.dot(adjbf_ref[...], innerbf_ref[...],
                 preferred_element_type=jnp.float32)
    support = (1.0 - _ALPHA) * hi + _ALPHA * h0_ref[...]
    out = jax.nn.relu(
        jnp.dot(support, weff_ref[...], preferred_element_type=jnp.float32))
    o_ref[...] = out.astype(o_ref.dtype)


def kernel(x, dia_len, topicLabel, adj, W0, b0, Wc):
    n, nfeat = x.shape
    nhid = W0.shape[0]
    nlayers = Wc.shape[0]
    bm = _BM if n % _BM == 0 else n
    grid = (n // bm,)
    bme = _BME if n % _BME == 0 else n

    h0, h0_bf = pl.pallas_call(
        _entry_kernel,
        grid=(n // bme,),
        in_specs=[
            pl.BlockSpec((bme, nfeat), lambda i: (i, 0)),
            pl.BlockSpec((nfeat, nhid), lambda i: (0, 0)),
            pl.BlockSpec((1, nhid), lambda i: (0, 0)),
        ],
        out_specs=[
            pl.BlockSpec((bme, nhid), lambda i: (i, 0)),
            pl.BlockSpec((bme, nhid), lambda i: (i, 0)),
        ],
        out_shape=[
            jax.ShapeDtypeStruct((n, nhid), jnp.float32),
            jax.ShapeDtypeStruct((n, nhid), jnp.bfloat16),
        ],
        compiler_params=pltpu.CompilerParams(
            dimension_semantics=("parallel",)),
    )(x, W0.T, b0.reshape(1, nhid))

    eye = jnp.eye(nhid, dtype=jnp.float32)

    def w_eff(i):
        theta = math.log(_LAMDA / (i + 1) + 1.0)
        return theta * Wc[i] + (1.0 - theta) * eye

    inner_bf, adj_bf = pl.pallas_call(
        _layer1_kernel,
        grid=grid,
        in_specs=[
            pl.BlockSpec((bm, n), lambda i: (i, 0)),
            pl.BlockSpec((n, nhid), lambda i: (0, 0)),
            pl.BlockSpec((bm, nhid), lambda i: (i, 0)),
            pl.BlockSpec((nhid, nhid), lambda i: (0, 0)),
        ],
        out_specs=[
            pl.BlockSpec((bm, nhid), lambda i: (i, 0)),
            pl.BlockSpec((bm, n), lambda i: (i, 0)),
        ],
        out_shape=[
            jax.ShapeDtypeStruct((n, nhid), jnp.bfloat16),
            jax.ShapeDtypeStruct((n, n), jnp.bfloat16),
        ],
        compiler_params=pltpu.CompilerParams(
            dimension_semantics=("parallel",)),
    )(adj, h0_bf, h0, w_eff(0))

    bm2 = _BM2 if n % _BM2 == 0 else bm
    grid2 = (n // bm2,)
    for i in range(1, nlayers):
        out_dtype = jnp.float32 if i == nlayers - 1 else jnp.bfloat16
        inner_bf = pl.pallas_call(
            _layer_kernel,
            grid=grid2,
            in_specs=[
                pl.BlockSpec((bm2, n), lambda i: (i, 0)),
                pl.BlockSpec((n, nhid), lambda i: (0, 0)),
                pl.BlockSpec((bm2, nhid), lambda i: (i, 0)),
                pl.BlockSpec((nhid, nhid), lambda i: (0, 0)),
            ],
            out_specs=pl.BlockSpec((bm2, nhid), lambda i: (i, 0)),
            out_shape=jax.ShapeDtypeStruct((n, nhid), out_dtype),
            compiler_params=pltpu.CompilerParams(
                dimension_semantics=("parallel",)),
        )(adj_bf, inner_bf, h0, w_eff(i))
    return inner_bf


# speedup vs baseline: 1.2679x; 1.0235x over previous
import math

import jax
import jax.numpy as jnp
from jax.experimental import pallas as pl
from jax.experimental.pallas import tpu as pltpu

_LAMDA = 0.5
_ALPHA = 0.1
_BM = 400
_BM2 = 1000
_BME = 2000


def _entry_kernel(x_ref, w0t_ref, b0_ref, o_ref, obf_ref):
    h = jax.nn.relu(
        jnp.dot(x_ref[...], w0t_ref[...], preferred_element_type=jnp.float32)
        + b0_ref[...]
    )
    o_ref[...] = h
    obf_ref[...] = h.astype(jnp.bfloat16)


def _layer1_kernel(adj_ref, innerbf_ref, h0_ref, weff_ref, o_ref, adjbf_ref):
    adj_bf = adj_ref[...].astype(jnp.bfloat16)
    adjbf_ref[...] = adj_bf
    hi = jnp.dot(adj_bf, innerbf_ref[...],
                 preferred_element_type=jnp.float32)
    support = (1.0 - _ALPHA) * hi + _ALPHA * h0_ref[...]
    o_ref[...] = jax.nn.relu(
        jnp.dot(support, weff_ref[...], preferred_element_type=jnp.float32)
    ).astype(jnp.bfloat16)


def _layer_kernel(adjbf_ref, innerbf_ref, h0_ref, weff_ref, o_ref):
    hi = jnp.dot(adjbf_ref[...], innerbf_ref[...],
                 preferred_element_type=jnp.float32)
    support = (1.0 - _ALPHA) * hi + _ALPHA * h0_ref[...]
    out = jax.nn.relu(
        jnp.dot(support, weff_ref[...], preferred_element_type=jnp.float32))
    o_ref[...] = out.astype(o_ref.dtype)


def kernel(x, dia_len, topicLabel, adj, W0, b0, Wc):
    n, nfeat = x.shape
    nhid = W0.shape[0]
    nlayers = Wc.shape[0]
    bm = _BM if n % _BM == 0 else n
    grid = (n // bm,)
    bme = _BME if n % _BME == 0 else n

    h0, h0_bf = pl.pallas_call(
        _entry_kernel,
        grid=(n // bme,),
        in_specs=[
            pl.BlockSpec((bme, nfeat), lambda i: (i, 0)),
            pl.BlockSpec((nfeat, nhid), lambda i: (0, 0)),
            pl.BlockSpec((1, nhid), lambda i: (0, 0)),
        ],
        out_specs=[
            pl.BlockSpec((bme, nhid), lambda i: (i, 0)),
            pl.BlockSpec((bme, nhid), lambda i: (i, 0)),
        ],
        out_shape=[
            jax.ShapeDtypeStruct((n, nhid), jnp.float32),
            jax.ShapeDtypeStruct((n, nhid), jnp.bfloat16),
        ],
        compiler_params=pltpu.CompilerParams(
            dimension_semantics=("parallel",)),
    )(x, W0.T, b0.reshape(1, nhid))

    eye = jnp.eye(nhid, dtype=jnp.float32)

    def w_eff(i):
        theta = math.log(_LAMDA / (i + 1) + 1.0)
        return theta * Wc[i] + (1.0 - theta) * eye

    inner_bf, adj_bf = pl.pallas_call(
        _layer1_kernel,
        grid=grid,
        in_specs=[
            pl.BlockSpec((bm, n), lambda i: (i, 0)),
            pl.BlockSpec((n, nhid), lambda i: (0, 0)),
            pl.BlockSpec((bm, nhid), lambda i: (i, 0)),
            pl.BlockSpec((nhid, nhid), lambda i: (0, 0)),
        ],
        out_specs=[
            pl.BlockSpec((bm, nhid), lambda i: (i, 0)),
            pl.BlockSpec((bm, n), lambda i: (i, 0)),
        ],
        out_shape=[
            jax.ShapeDtypeStruct((n, nhid), jnp.bfloat16),
            jax.ShapeDtypeStruct((n, n), jnp.bfloat16),
        ],
        compiler_params=pltpu.CompilerParams(
            dimension_semantics=("parallel",)),
    )(adj, h0_bf, h0, w_eff(0))

    bm2 = _BM2 if n % _BM2 == 0 else bm
    grid2 = (n // bm2,)
    for i in range(1, nlayers):
        out_dtype = jnp.float32 if i == nlayers - 1 else jnp.bfloat16
        inner_bf = pl.pallas_call(
            _layer_kernel,
            grid=grid2,
            in_specs=[
                pl.BlockSpec((bm2, n), lambda i: (i, 0)),
                pl.BlockSpec((n, nhid), lambda i: (0, 0)),
                pl.BlockSpec((bm2, nhid), lambda i: (i, 0)),
                pl.BlockSpec((nhid, nhid), lambda i: (0, 0)),
            ],
            out_specs=pl.BlockSpec((bm2, nhid), lambda i: (i, 0)),
            out_shape=jax.ShapeDtypeStruct((n, nhid), out_dtype),
            compiler_params=pltpu.CompilerParams(
                dimension_semantics=("parallel",)),
        )(adj_bf, inner_bf, h0, w_eff(i))
    return inner_bf
